# Initial kernel scaffold; baseline (speedup 1.0000x reference)
import jax
import jax.numpy as jnp
from jax.experimental import pallas as pl


def kernel(x, tables):
    raise NotImplementedError("write your pallas kernel here")


# speedup vs baseline: 35.2491x; 35.2491x over previous
import functools

import numpy as np
import jax
import jax.numpy as jnp
from jax import lax
from jax.experimental import pallas as pl
from jax.experimental.pallas import tpu as pltpu
from jax.experimental.pallas import tpu_sc as plsc

_N_LEVELS = 16
_TS = 2 ** 19
_MASK = _TS - 1
_PRIME = np.int32(np.uint32(2654435761).astype(np.int64) - (1 << 32))
_PAIRS = ((0, 1), (0, 2), (1, 2))
_NCOMBO = _N_LEVELS * 3

_B = 1048576
_NT = 32
_PPT = _B // _NT
_C = 256
_G = _C // 16
_NCHUNK = _PPT // _C
_NBLK = 4 * _C // 128


def _combo_consts():
    base = 16.0
    growth = np.exp((np.log(512.0) - np.log(16.0)) / (_N_LEVELS - 1))
    invg = np.zeros((_NCOMBO, 16), np.float32)
    icon = np.zeros((3, _NCOMBO, 16), np.int32)
    for i in range(_N_LEVELS):
        res = float(np.floor(base * (growth ** i)))
        for j, (a, b) in enumerate(_PAIRS):
            c = 3 * i + j
            invg[c, :] = np.float32(res / 2.0)
            icon[0, c, :] = (j * _N_LEVELS + i) * _TS
            icon[1, c, :] = a
            icon[2, c, :] = b
    return invg.reshape(-1), icon.reshape(-1)


_INVG_NP, _ICON_NP = _combo_consts()


def _tile_body(xT, tflat, invg, icon, out,
               x_v, out_v, idx_v, rows_v, w_v, invg_v, icon_v, sems):
    wid = lax.axis_index("s") * 2 + lax.axis_index("c")
    iota = lax.iota(jnp.int32, 16)
    pltpu.sync_copy(invg, invg_v)
    pltpu.sync_copy(icon, icon_v)

    def fire(c, slot):
        igv = invg_v[pl.ds(c * 16, 16)]
        hbase = icon_v[pl.ds(c * 16, 16)]
        av = icon_v[pl.ds((_NCOMBO + c) * 16, 16)]
        bv = icon_v[pl.ds((2 * _NCOMBO + c) * 16, 16)]
        conda = av == 0
        condb = bv == 1
        wbase = slot * (2 * _C)
        ibase = slot * (2 * 4 * _C)

        def grp(g, _):
            o = g * 16
            x0 = x_v[pl.ds(o, 16)]
            x1 = x_v[pl.ds(_C + o, 16)]
            x2 = x_v[pl.ds(2 * _C + o, 16)]
            xa = jnp.where(conda, x0, x1)
            xb = jnp.where(condb, x1, x2)
            ta = (xa + 1.0) * igv
            tb = (xb + 1.0) * igv
            ia0 = ta.astype(jnp.int32)
            ib0 = tb.astype(jnp.int32)
            w_v[pl.ds(wbase + o, 16)] = ta - ia0.astype(jnp.float32)
            w_v[pl.ds(wbase + _C + o, 16)] = tb - ib0.astype(jnp.float32)
            ia1 = ia0 + 1
            hb0 = ib0 * _PRIME
            hb1 = hb0 + _PRIME
            for k, (pa, pb) in enumerate(((ia0, hb0), (ia0, hb1),
                                          (ia1, hb0), (ia1, hb1))):
                f0 = (((pa ^ pb) & _MASK) + hbase) * 2
                idx_v[pl.ds(ibase + k * _C + o, 16)] = f0
                idx_v[pl.ds(ibase + 4 * _C + k * _C + o, 16)] = f0 + 1
            return 0

        lax.fori_loop(0, _G, grp, 0)
        for f in range(2):
            for blk in range(_NBLK):
                off = ibase + f * (4 * _C) + blk * 128
                pltpu.async_copy(
                    tflat.at[idx_v.at[pl.ds(off, 128)]],
                    rows_v.at[pl.ds(off, 128)],
                    sems.at[slot])

    def drain_blend(c, slot):
        wbase = slot * (2 * _C)
        ibase = slot * (2 * 4 * _C)
        for f in range(2):
            for blk in range(_NBLK):
                off = ibase + f * (4 * _C) + blk * 128
                pltpu.make_async_copy(
                    tflat.at[idx_v.at[pl.ds(off, 128)]],
                    rows_v.at[pl.ds(off, 128)],
                    sems.at[slot]).wait()
        colbase = 2 * c

        def grp(g, _):
            o = g * 16
            w0 = w_v[pl.ds(wbase + o, 16)]
            w1 = w_v[pl.ds(wbase + _C + o, 16)]
            u0 = 1.0 - w0
            u1 = 1.0 - w1
            obase = (g * 16 + iota) * 96 + colbase
            for f in range(2):
                rb = ibase + f * (4 * _C) + o
                e00 = rows_v[pl.ds(rb, 16)]
                e01 = rows_v[pl.ds(rb + _C, 16)]
                e10 = rows_v[pl.ds(rb + 2 * _C, 16)]
                e11 = rows_v[pl.ds(rb + 3 * _C, 16)]
                v0 = e00 * u0 + e10 * w0
                v1 = e01 * u0 + e11 * w0
                plsc.store_scatter(out_v, [obase + f], v0 * u1 + v1 * w1)
            return 0

        lax.fori_loop(0, _G, grp, 0)

    def chunk_body(ch, _):
        base = wid * _PPT + ch * _C
        for d in range(3):
            pltpu.sync_copy(xT.at[pl.ds(d * _B + base, _C)],
                            x_v.at[pl.ds(d * _C, _C)])

        def combo_body(c, _):
            slot = jnp.bitwise_and(c, 1)
            pslot = jnp.bitwise_and(c - 1, 1)

            @pl.when(c < _NCOMBO)
            def _():
                fire(c, slot)

            @pl.when(c > 0)
            def _():
                drain_blend(c - 1, pslot)
            return 0

        lax.fori_loop(0, _NCOMBO + 1, combo_body, 0)
        pltpu.sync_copy(out_v, out.at[pl.ds(base * 96, _C * 96)])
        return 0

    lax.fori_loop(0, _NCHUNK, chunk_body, 0)


@jax.jit
def kernel(x, tables):
    xT = x.T.reshape(-1)
    tflat = tables.reshape(-1)
    invg = jnp.asarray(_INVG_NP)
    icon = jnp.asarray(_ICON_NP)
    mesh = plsc.VectorSubcoreMesh(core_axis_name="c", subcore_axis_name="s")
    run = pl.kernel(
        _tile_body,
        out_type=jax.ShapeDtypeStruct((_B * 96,), jnp.float32),
        mesh=mesh,
        compiler_params=pltpu.CompilerParams(needs_layout_passes=False),
        scratch_types=[
            pltpu.VMEM((3 * _C,), jnp.float32),
            pltpu.VMEM((_C * 96,), jnp.float32),
            pltpu.VMEM((2 * 2 * 4 * _C,), jnp.int32),
            pltpu.VMEM((2 * 2 * 4 * _C,), jnp.float32),
            pltpu.VMEM((2 * 2 * _C,), jnp.float32),
            pltpu.VMEM((_NCOMBO * 16,), jnp.float32),
            pltpu.VMEM((3 * _NCOMBO * 16,), jnp.int32),
            pltpu.SemaphoreType.DMA((2,)),
        ],
    )
    return run(xT, tflat, invg, icon).reshape(_B, 96)


# speedup vs baseline: 36.3114x; 1.0301x over previous
import functools

import numpy as np
import jax
import jax.numpy as jnp
from jax import lax
from jax.experimental import pallas as pl
from jax.experimental.pallas import tpu as pltpu
from jax.experimental.pallas import tpu_sc as plsc

_N_LEVELS = 16
_TS = 2 ** 19
_MASK = _TS - 1
_PRIME = np.int32(np.uint32(2654435761).astype(np.int64) - (1 << 32))
_PAIRS = ((0, 1), (0, 2), (1, 2))
_NCOMBO = _N_LEVELS * 3

_B = 1048576
_NT = 32
_PPT = _B // _NT
_C = 256
_G = _C // 16
_NCHUNK = _PPT // _C


def _combo_consts():
    base = 16.0
    growth = np.exp((np.log(512.0) - np.log(16.0)) / (_N_LEVELS - 1))
    invg = np.zeros((_NCOMBO, 16), np.float32)
    icon = np.zeros((3, _NCOMBO, 16), np.int32)
    for i in range(_N_LEVELS):
        res = float(np.floor(base * (growth ** i)))
        for j, (a, b) in enumerate(_PAIRS):
            c = 3 * i + j
            invg[c, :] = np.float32(res / 2.0)
            icon[0, c, :] = (j * _N_LEVELS + i) * _TS
            icon[1, c, :] = a
            icon[2, c, :] = b
    return invg.reshape(-1), icon.reshape(-1)


_INVG_NP, _ICON_NP = _combo_consts()


def _tile_body(xf, tflat, invg, icon, out,
               x_v, out_v, idxA_v, idxB_v, rowsA_v, rowsB_v, wA_v, wB_v,
               invg_v, icon_v, sems):
    wid = lax.axis_index("s") * 2 + lax.axis_index("c")
    iota = lax.iota(jnp.int32, 16)
    iota3 = iota * 3
    zeros16 = jnp.zeros((16,), jnp.int32)
    ones16 = zeros16 + 1
    pltpu.sync_copy(invg, invg_v)
    pltpu.sync_copy(icon, icon_v)

    def fire(c, idx_v, w_v, rows_v, sem_i):
        igv = invg_v[pl.ds(c * 16, 16)]
        hbase = icon_v[pl.ds(c * 16, 16)]
        av = icon_v[pl.ds((_NCOMBO + c) * 16, 16)]
        bv = icon_v[pl.ds((2 * _NCOMBO + c) * 16, 16)]
        conda = av == 0
        condb = bv == 1

        def grp(g, _):
            o = g * 16
            x0 = plsc.load_gather(x_v, [iota3 + 3 * o])
            x1 = plsc.load_gather(x_v, [iota3 + (3 * o + 1)])
            x2 = plsc.load_gather(x_v, [iota3 + (3 * o + 2)])
            xa = jnp.where(conda, x0, x1)
            xb = jnp.where(condb, x1, x2)
            ta = (xa + 1.0) * igv
            tb = (xb + 1.0) * igv
            ia0 = ta.astype(jnp.int32)
            ib0 = tb.astype(jnp.int32)
            w_v[pl.ds(o, 16)] = ta - ia0.astype(jnp.float32)
            w_v[pl.ds(_C + o, 16)] = tb - ib0.astype(jnp.float32)
            ia1 = ia0 + 1
            hb0 = ib0 * _PRIME
            hb1 = hb0 + _PRIME
            for k, (pa, pb) in enumerate(((ia0, hb0), (ia0, hb1),
                                          (ia1, hb0), (ia1, hb1))):
                f0 = (((pa ^ pb) & _MASK) + hbase) * 2
                idx_v[pl.ds(k * _C + o, 16)] = f0
                idx_v[pl.ds(4 * _C + k * _C + o, 16)] = f0 + 1
            return 0

        lax.fori_loop(0, _G, grp, 0)
        for f in range(2):
            pltpu.async_copy(
                tflat.at[idx_v.at[pl.ds(f * 4 * _C, 4 * _C)]],
                rows_v.at[pl.ds(f * 4 * _C, 4 * _C)],
                sems.at[sem_i])

    def drain_blend(c, idx_v, w_v, rows_v, sem_i):
        for f in range(2):
            pltpu.make_async_copy(
                tflat.at[idx_v.at[pl.ds(f * 4 * _C, 4 * _C)]],
                rows_v.at[pl.ds(f * 4 * _C, 4 * _C)],
                sems.at[sem_i]).wait()
        colbase = 2 * c

        def grp(g, _):
            o = g * 16
            w0 = w_v[pl.ds(o, 16)]
            w1 = w_v[pl.ds(_C + o, 16)]
            u0 = 1.0 - w0
            u1 = 1.0 - w1
            obase = (g * 16 + iota) * 96 + colbase
            for f in range(2):
                rb = f * 4 * _C + o
                e00 = rows_v[pl.ds(rb, 16)]
                e01 = rows_v[pl.ds(rb + _C, 16)]
                e10 = rows_v[pl.ds(rb + 2 * _C, 16)]
                e11 = rows_v[pl.ds(rb + 3 * _C, 16)]
                v0 = e00 * u0 + e10 * w0
                v1 = e01 * u0 + e11 * w0
                plsc.store_scatter(out_v, [obase + f], v0 * u1 + v1 * w1)
            return 0

        lax.fori_loop(0, _G, grp, 0)

    def chunk_body(ch, _):
        base = wid * _PPT + ch * _C
        pltpu.sync_copy(xf.at[pl.ds(base * 3, 3 * _C)], x_v)
        fire(jnp.int32(0), idxA_v, wA_v, rowsA_v, 0)

        def combo_body(t, _):
            c = 2 * t

            @pl.when(c + 1 < _NCOMBO)
            def _():
                fire(c + 1, idxB_v, wB_v, rowsB_v, 1)

            drain_blend(c, idxA_v, wA_v, rowsA_v, 0)

            @pl.when(c + 2 < _NCOMBO)
            def _():
                fire(c + 2, idxA_v, wA_v, rowsA_v, 0)

            @pl.when(c + 1 < _NCOMBO)
            def _():
                drain_blend(c + 1, idxB_v, wB_v, rowsB_v, 1)
            return 0

        lax.fori_loop(0, _NCOMBO // 2, combo_body, 0)
        pltpu.sync_copy(out_v, out.at[pl.ds(base * 96, _C * 96)])
        return 0

    lax.fori_loop(0, _NCHUNK, chunk_body, 0)


@jax.jit
def kernel(x, tables):
    xf = x.reshape(-1)
    tflat = tables.reshape(-1)
    invg = jnp.asarray(_INVG_NP)
    icon = jnp.asarray(_ICON_NP)
    mesh = plsc.VectorSubcoreMesh(core_axis_name="c", subcore_axis_name="s")
    run = pl.kernel(
        _tile_body,
        out_type=jax.ShapeDtypeStruct((_B * 96,), jnp.float32),
        mesh=mesh,
        compiler_params=pltpu.CompilerParams(needs_layout_passes=False),
        scratch_types=[
            pltpu.VMEM((3 * _C,), jnp.float32),
            pltpu.VMEM((_C * 96,), jnp.float32),
            pltpu.VMEM((2 * 4 * _C,), jnp.int32),
            pltpu.VMEM((2 * 4 * _C,), jnp.int32),
            pltpu.VMEM((2 * 4 * _C,), jnp.float32),
            pltpu.VMEM((2 * 4 * _C,), jnp.float32),
            pltpu.VMEM((2 * _C,), jnp.float32),
            pltpu.VMEM((2 * _C,), jnp.float32),
            pltpu.VMEM((_NCOMBO * 16,), jnp.float32),
            pltpu.VMEM((3 * _NCOMBO * 16,), jnp.int32),
            pltpu.SemaphoreType.DMA((2,)),
        ],
    )
    return run(xf, tflat, invg, icon).reshape(_B, 96)


# speedup vs baseline: 71.8411x; 1.9785x over previous
import functools

import numpy as np
import jax
import jax.numpy as jnp
from jax import lax
from jax.experimental import pallas as pl
from jax.experimental.pallas import tpu as pltpu
from jax.experimental.pallas import tpu_sc as plsc

_N_LEVELS = 16
_TS = 2 ** 19
_MASK = _TS - 1
_PRIME = np.int32(np.uint32(2654435761).astype(np.int64) - (1 << 32))
_PAIRS = ((0, 1), (0, 2), (1, 2))
_NCOMBO = _N_LEVELS * 3

_B = 1048576
_NT = 32
_PPT = _B // _NT
_C = 256
_G = _C // 16
_NCHUNK = _PPT // _C


def _combo_consts():
    base = 16.0
    growth = np.exp((np.log(512.0) - np.log(16.0)) / (_N_LEVELS - 1))
    invg = np.zeros((_NCOMBO, 16), np.float32)
    icon = np.zeros((3, _NCOMBO, 16), np.int32)
    for i in range(_N_LEVELS):
        res = float(np.floor(base * (growth ** i)))
        for j, (a, b) in enumerate(_PAIRS):
            c = 3 * i + j
            invg[c, :] = np.float32(res / 2.0)
            icon[0, c, :] = j * _N_LEVELS + i
            icon[1, c, :] = a
            icon[2, c, :] = b
    return invg.reshape(-1), icon.reshape(-1)


_INVG_NP, _ICON_NP = _combo_consts()


def _tile_body(xf, tflat, invg, icon, out,
               x_v, out_v, idxA_v, idxB_v, rowsA_v, rowsB_v, wA_v, wB_v,
               invg_v, icon_v, sems):
    wid = lax.axis_index("s") * 2 + lax.axis_index("c")
    iota = lax.iota(jnp.int32, 16)
    iota3 = iota * 3
    zeros16 = jnp.zeros((16,), jnp.int32)
    ones16 = zeros16 + 1
    pltpu.sync_copy(invg, invg_v)
    pltpu.sync_copy(icon, icon_v)

    def fire(c, idx_v, w_v, rows_v, sem_i):
        igv = invg_v[pl.ds(c * 16, 16)]
        cc = icon_v[pl.ds(c * 16, 16)]
        av = icon_v[pl.ds((_NCOMBO + c) * 16, 16)]
        bv = icon_v[pl.ds((2 * _NCOMBO + c) * 16, 16)]
        conda = av == 0
        condb = bv == 1

        def grp(g, _):
            o = g * 16
            x0 = plsc.load_gather(x_v, [iota3 + 3 * o])
            x1 = plsc.load_gather(x_v, [iota3 + (3 * o + 1)])
            x2 = plsc.load_gather(x_v, [iota3 + (3 * o + 2)])
            xa = jnp.where(conda, x0, x1)
            xb = jnp.where(condb, x1, x2)
            ta = (xa + 1.0) * igv
            tb = (xb + 1.0) * igv
            ia0 = ta.astype(jnp.int32)
            ib0 = tb.astype(jnp.int32)
            w_v[pl.ds(o, 16)] = ta - ia0.astype(jnp.float32)
            w_v[pl.ds(_C + o, 16)] = tb - ib0.astype(jnp.float32)
            ia1 = ia0 + 1
            hb0 = ib0 * _PRIME
            hb1 = hb0 + _PRIME
            for k, (pa, pb) in enumerate(((ia0, hb0), (ia0, hb1),
                                          (ia1, hb0), (ia1, hb1))):
                f0 = ((pa ^ pb) & _MASK) * 2
                idx_v[pl.ds(k * _C + o, 16)] = f0
                idx_v[pl.ds(4 * _C + k * _C + o, 16)] = f0 + 1
            return 0

        lax.fori_loop(0, _G, grp, 0)
        ccs = jnp.min(cc)
        for f in range(2):
            pltpu.async_copy(
                tflat.at[ccs].at[idx_v.at[pl.ds(f * 4 * _C, 4 * _C)]],
                rows_v.at[pl.ds(f * 4 * _C, 4 * _C)],
                sems.at[sem_i])

    def drain_blend(c, idx_v, w_v, rows_v, sem_i):
        for f in range(2):
            pltpu.make_async_copy(
                tflat.at[0].at[idx_v.at[pl.ds(f * 4 * _C, 4 * _C)]],
                rows_v.at[pl.ds(f * 4 * _C, 4 * _C)],
                sems.at[sem_i]).wait()
        colbase = 2 * c

        def grp(g, _):
            o = g * 16
            w0 = w_v[pl.ds(o, 16)]
            w1 = w_v[pl.ds(_C + o, 16)]
            u0 = 1.0 - w0
            u1 = 1.0 - w1
            obase = (g * 16 + iota) * 96 + colbase
            for f in range(2):
                rb = f * 4 * _C + o
                e00 = rows_v[pl.ds(rb, 16)]
                e01 = rows_v[pl.ds(rb + _C, 16)]
                e10 = rows_v[pl.ds(rb + 2 * _C, 16)]
                e11 = rows_v[pl.ds(rb + 3 * _C, 16)]
                v0 = e00 * u0 + e10 * w0
                v1 = e01 * u0 + e11 * w0
                plsc.store_scatter(out_v, [obase + f], v0 * u1 + v1 * w1)
            return 0

        lax.fori_loop(0, _G, grp, 0)

    def chunk_body(ch, _):
        base = wid * _PPT + ch * _C
        pltpu.sync_copy(xf.at[pl.ds(base * 3, 3 * _C)], x_v)
        fire(jnp.int32(0), idxA_v, wA_v, rowsA_v, 0)

        def combo_body(t, _):
            c = 2 * t

            @pl.when(c + 1 < _NCOMBO)
            def _():
                fire(c + 1, idxB_v, wB_v, rowsB_v, 1)

            drain_blend(c, idxA_v, wA_v, rowsA_v, 0)

            @pl.when(c + 2 < _NCOMBO)
            def _():
                fire(c + 2, idxA_v, wA_v, rowsA_v, 0)

            @pl.when(c + 1 < _NCOMBO)
            def _():
                drain_blend(c + 1, idxB_v, wB_v, rowsB_v, 1)
            return 0

        lax.fori_loop(0, _NCOMBO // 2, combo_body, 0)
        pltpu.sync_copy(out_v, out.at[pl.ds(base * 96, _C * 96)])
        return 0

    lax.fori_loop(0, _NCHUNK, chunk_body, 0)


@jax.jit
def kernel(x, tables):
    xf = x.reshape(-1)
    tflat = tables.reshape(3 * _N_LEVELS, _TS * 2)
    invg = jnp.asarray(_INVG_NP)
    icon = jnp.asarray(_ICON_NP)
    mesh = plsc.VectorSubcoreMesh(core_axis_name="c", subcore_axis_name="s")
    run = pl.kernel(
        _tile_body,
        out_type=jax.ShapeDtypeStruct((_B * 96,), jnp.float32),
        mesh=mesh,
        compiler_params=pltpu.CompilerParams(needs_layout_passes=False,
                                             use_tc_tiling_on_sc=False),
        scratch_types=[
            pltpu.VMEM((3 * _C,), jnp.float32),
            pltpu.VMEM((_C * 96,), jnp.float32),
            pltpu.VMEM((2 * 4 * _C,), jnp.int32),
            pltpu.VMEM((2 * 4 * _C,), jnp.int32),
            pltpu.VMEM((2 * 4 * _C,), jnp.float32),
            pltpu.VMEM((2 * 4 * _C,), jnp.float32),
            pltpu.VMEM((2 * _C,), jnp.float32),
            pltpu.VMEM((2 * _C,), jnp.float32),
            pltpu.VMEM((_NCOMBO * 16,), jnp.float32),
            pltpu.VMEM((3 * _NCOMBO * 16,), jnp.int32),
            pltpu.SemaphoreType.DMA((2,)),
        ],
    )
    return run(xf, tflat, invg, icon).reshape(_B, 96)
